# baseline (speedup 1.0000x reference)
import jax
import jax.numpy as jnp
from jax import lax
from jax.experimental import pallas as pl
from jax.experimental.pallas import tpu as pltpu
from jax.experimental.pallas import tpu_sc as plsc

NUM_CLASSES = 5
LAT = 16
BATCH = 16384
NC, NS, L = 2, 16, 16
NW = NC * NS
BPW = BATCH // NW
CHUNK = 128
NCHUNK = BPW // CHUNK


def _body(user_hbm, movie_hbm, utmf_hbm, mtmf_hbm, ut_hbm, mt_hbm,
          fcw_hbm, fcb_hbm, out_hbm,
          idx_u, idx_m, umf_v, mmf_v, u_v, m_v, w_v, b_v, out_v, sem):
    wid = lax.axis_index("s") * NC + lax.axis_index("c")
    row0 = wid * NCHUNK

    pltpu.sync_copy(user_hbm.at[pl.ds(row0, NCHUNK)], idx_u)
    pltpu.sync_copy(movie_hbm.at[pl.ds(row0, NCHUNK)], idx_m)
    pltpu.sync_copy(fcw_hbm, w_v)
    pltpu.sync_copy(fcb_hbm, b_v)

    copies = []
    for j in range(NCHUNK):
        dst = pl.ds(j * CHUNK, CHUNK)
        copies.append(pltpu.async_copy(utmf_hbm.at[idx_u.at[j]], umf_v.at[dst], sem))
        copies.append(pltpu.async_copy(mtmf_hbm.at[idx_m.at[j]], mmf_v.at[dst], sem))
        copies.append(pltpu.async_copy(ut_hbm.at[idx_u.at[j]], u_v.at[dst], sem))
        copies.append(pltpu.async_copy(mt_hbm.at[idx_m.at[j]], m_v.at[dst], sem))
    for c in copies:
        c.wait()

    lane = lax.iota(jnp.int32, L)
    bias = b_v[...]

    def elem(b, carry):
        mf = umf_v[b, :] * mmf_v[b, :]
        u = u_v[b, :]
        m = m_v[b, :]
        acc = bias
        for c in range(NUM_CLASSES):
            t = (mf * w_v[c, 0:LAT] + u * w_v[c, LAT:2 * LAT]
                 + m * w_v[c, 2 * LAT:3 * LAT])
            s = jnp.sum(t)
            acc = jnp.where(lane == c, acc + s, acc)
        out_v[b, :] = acc
        return carry

    lax.fori_loop(0, BPW, elem, 0)

    pltpu.sync_copy(out_v, out_hbm.at[pl.ds(wid * BPW, BPW)])


def kernel(user, movie, user_table_mf, movie_table_mf, user_table,
           movie_table, fc_w, fc_b):
    user2 = user.reshape(NW * NCHUNK, CHUNK)
    movie2 = movie.reshape(NW * NCHUNK, CHUNK)
    fcb_pad = jnp.pad(fc_b, (0, L - NUM_CLASSES))
    run = pl.kernel(
        _body,
        out_type=jax.ShapeDtypeStruct((BATCH, L), jnp.float32),
        mesh=plsc.VectorSubcoreMesh(core_axis_name="c", subcore_axis_name="s"),
        compiler_params=pltpu.CompilerParams(needs_layout_passes=False,
                                             use_tc_tiling_on_sc=False),
        scratch_types=[
            pltpu.VMEM((NCHUNK, CHUNK), jnp.int32),
            pltpu.VMEM((NCHUNK, CHUNK), jnp.int32),
            pltpu.VMEM((BPW, LAT), jnp.float32),
            pltpu.VMEM((BPW, LAT), jnp.float32),
            pltpu.VMEM((BPW, LAT), jnp.float32),
            pltpu.VMEM((BPW, LAT), jnp.float32),
            pltpu.VMEM((NUM_CLASSES, 3 * LAT), jnp.float32),
            pltpu.VMEM((L,), jnp.float32),
            pltpu.VMEM((BPW, L), jnp.float32),
            pltpu.SemaphoreType.DMA,
        ],
    )
    out_pad = run(user2, movie2, user_table_mf, movie_table_mf, user_table,
                  movie_table, fc_w, fcb_pad)
    return out_pad[:, :NUM_CLASSES]
